# Initial kernel scaffold; baseline (speedup 1.0000x reference)
import jax
import jax.numpy as jnp
from jax.experimental import pallas as pl


def kernel(edge_index, edge_weight, X, W):
    raise NotImplementedError("write your pallas kernel here")


# speedup vs baseline: 4.3968x; 4.3968x over previous
import functools

import jax
import jax.numpy as jnp
from jax import lax
from jax.experimental import pallas as pl
from jax.experimental.pallas import tpu as pltpu
from jax.experimental.pallas import tpu_sc as plsc

N = 10000
E = 320000
M = 128
H = 128

NC = 2
NS = 16
HC = M // NC
EPT = E // NS
B = 80
NB = EPT // B
LANES = 16
HCHUNKS = HC // LANES
G16 = B // LANES
ROWS_PER_TILE = 632


def _sc_aggregate_body(src_hbm, dst_hbm, w_hbm, xt_hbm, part_hbm,
                       src_v, dst_v, w_v, g0, g1, acc, sem0, sem1):
    c = lax.axis_index("c")
    s = lax.axis_index("s")

    pltpu.sync_copy(src_hbm.at[s], src_v)
    pltpu.sync_copy(dst_hbm.at[s], dst_v)
    pltpu.sync_copy(w_hbm.at[s], w_v)

    zeros16 = jnp.zeros((LANES,), jnp.float32)

    def _zrow(i, _):
        for h in range(HCHUNKS):
            g0[i, pl.ds(h * LANES, LANES)] = zeros16
        return 0

    lax.fori_loop(0, B, _zrow, 0)
    base_row = pl.multiple_of(
        jnp.minimum(s * ROWS_PER_TILE, N - ROWS_PER_TILE), 8)
    full, rem = divmod(ROWS_PER_TILE, B)
    for k in range(full):
        pltpu.sync_copy(g0, acc.at[pl.ds(base_row + k * B, B)])
    if rem:
        pltpu.sync_copy(g0.at[pl.ds(0, rem)],
                        acc.at[pl.ds(base_row + full * B, rem)])
    plsc.subcore_barrier()

    xc_hbm = xt_hbm.at[c]

    def _scale(buf, j):
        def _grp(g, _):
            wvec = w_v[j, pl.ds(g * LANES, LANES)]
            rbase = g * LANES
            for r in range(LANES):
                wr = wvec[r]
                row = rbase + r
                for h in range(HCHUNKS):
                    sl = pl.ds(h * LANES, LANES)
                    buf[row, sl] = buf[row, sl] * wr
            return 0

        lax.fori_loop(0, G16, _grp, 0)

    pltpu.async_copy(xc_hbm.at[src_v.at[0]], g0, sem0)

    def _step(jj, _):
        j = jj * 2
        pltpu.make_async_copy(xc_hbm.at[src_v.at[j]], g0, sem0).wait()
        pltpu.async_copy(xc_hbm.at[src_v.at[j + 1]], g1, sem1)
        _scale(g0, j)
        pltpu.sync_copy(g0, acc.at[dst_v.at[j]], add=True)

        pltpu.make_async_copy(xc_hbm.at[src_v.at[j + 1]], g1, sem1).wait()

        @pl.when(j + 2 < NB)
        def _():
            pltpu.async_copy(xc_hbm.at[src_v.at[j + 2]], g0, sem0)

        _scale(g1, j + 1)
        pltpu.sync_copy(g1, acc.at[dst_v.at[j + 1]], add=True)
        return 0

    lax.fori_loop(0, NB // 2, _step, 0)

    plsc.subcore_barrier()
    rows = pl.ds(base_row, ROWS_PER_TILE)
    pltpu.sync_copy(acc.at[rows], part_hbm.at[c].at[rows])


_sc_aggregate = functools.partial(
    pl.kernel,
    out_type=jax.ShapeDtypeStruct((NC, N, HC), jnp.float32),
    mesh=plsc.VectorSubcoreMesh(core_axis_name="c", subcore_axis_name="s"),
    compiler_params=pltpu.CompilerParams(use_tc_tiling_on_sc=False),
    scratch_types=[
        pltpu.VMEM((NB, B), jnp.int32),
        pltpu.VMEM((NB, B), jnp.int32),
        pltpu.VMEM((NB, B), jnp.float32),
        pltpu.VMEM((B, HC), jnp.float32),
        pltpu.VMEM((B, HC), jnp.float32),
        pltpu.VMEM_SHARED((N, HC), jnp.float32),
        pltpu.SemaphoreType.DMA,
        pltpu.SemaphoreType.DMA,
    ],
)(_sc_aggregate_body)


BM = 2000


def _tc_combine_body(p_ref, w_ref, o_ref):
    acc = (jnp.dot(p_ref[0], w_ref[:HC],
                   preferred_element_type=jnp.float32) +
           jnp.dot(p_ref[1], w_ref[HC:],
                   preferred_element_type=jnp.float32))
    o_ref[...] = jnp.maximum(acc, 0.0)


def _tc_combine(partials, W):
    return pl.pallas_call(
        _tc_combine_body,
        grid=(N // BM,),
        in_specs=[
            pl.BlockSpec((NC, BM, HC), lambda i: (0, i, 0)),
            pl.BlockSpec((M, H), lambda i: (0, 0)),
        ],
        out_specs=pl.BlockSpec((BM, H), lambda i: (i, 0)),
        out_shape=jax.ShapeDtypeStruct((N, H), jnp.float32),
    )(partials, W)


@jax.jit
def kernel(edge_index, edge_weight, X, W):
    dst = edge_index[0].reshape(NS, NB, B)
    src = edge_index[1].reshape(NS, NB, B)
    w = edge_weight.reshape(NS, NB, B)
    xt = X.reshape(N, NC, HC).swapaxes(0, 1)
    partials = _sc_aggregate(src, dst, w, xt)
    return _tc_combine(partials, W)


# speedup vs baseline: 8.2741x; 1.8819x over previous
import functools

import jax
import jax.numpy as jnp
from jax import lax
from jax.experimental import pallas as pl
from jax.experimental.pallas import tpu as pltpu
from jax.experimental.pallas import tpu_sc as plsc

N = 10000
E = 320000
M = 128
H = 128

NC = 2
NS = 16
HC = M // NC
EPT = E // NS
B = 80
NB = EPT // B
LANES = 16
HCHUNKS = HC // LANES
G16 = B // LANES
ROWS_PER_TILE = 632


NBUF = 4
NITER = 62


def _sc_aggregate_body(src_hbm, dst_hbm, w_hbm, xt_hbm, part_hbm,
                       src_v, dst_v, w_v, bufs, acc, gsems, ssems):
    c = lax.axis_index("c")
    s = lax.axis_index("s")

    pltpu.sync_copy(src_hbm.at[s], src_v)
    pltpu.sync_copy(dst_hbm.at[s], dst_v)
    pltpu.sync_copy(w_hbm.at[s], w_v)

    zeros16 = jnp.zeros((LANES,), jnp.float32)
    g0 = bufs[0]

    def _zrow(i, _):
        for h in range(HCHUNKS):
            g0[i, pl.ds(h * LANES, LANES)] = zeros16
        return 0

    lax.fori_loop(0, B, _zrow, 0)
    base_row = pl.multiple_of(
        jnp.minimum(s * ROWS_PER_TILE, N - ROWS_PER_TILE), 8)
    full, rem = divmod(ROWS_PER_TILE, B)
    for k in range(full):
        pltpu.sync_copy(g0, acc.at[pl.ds(base_row + k * B, B)])
    if rem:
        pltpu.sync_copy(g0.at[pl.ds(0, rem)],
                        acc.at[pl.ds(base_row + full * B, rem)])
    plsc.subcore_barrier()

    xc_hbm = xt_hbm.at[c]

    def _scale(buf, j):
        for g in range(G16):
            wvec = w_v[j, pl.ds(g * LANES, LANES)]
            for r in range(LANES):
                wr = wvec[r]
                row = g * LANES + r
                for h in range(HCHUNKS):
                    sl = pl.ds(h * LANES, LANES)
                    buf[row, sl] = buf[row, sl] * wr

    def _gather(p, j):
        pltpu.async_copy(xc_hbm.at[src_v.at[j]], bufs[p], gsems[p])

    def _gather_wait(p, j):
        pltpu.make_async_copy(xc_hbm.at[src_v.at[j]], bufs[p], gsems[p]).wait()

    def _scatter(p, j):
        pltpu.async_copy(bufs[p], acc.at[dst_v.at[j]], ssems[p], add=True)

    def _scatter_wait(p, j):
        pltpu.make_async_copy(bufs[p], acc.at[dst_v.at[j]], ssems[p]).wait()

    _gather(0, 0)
    _gather(1, 1)

    def _step(jj, _):
        j0 = jj * NBUF
        for p in range(NBUF):
            j = j0 + p
            q = (p + 2) % NBUF
            _gather_wait(p, j)
            _scale(bufs[p], j)
            _scatter(p, j)
            if p < 2:
                @pl.when(jj > 0)
                def _():
                    _scatter_wait(q, j - 2)
                    _gather(q, j + 2)

                @pl.when(jj == 0)
                def _():
                    _gather(q, j + 2)
            else:
                _scatter_wait(q, j - 2)
                _gather(q, j + 2)
        return 0

    lax.fori_loop(0, NITER, _step, 0)

    jE = NBUF * NITER
    _gather_wait(0, jE)
    _scale(bufs[0], jE)
    _scatter(0, jE)
    _gather_wait(1, jE + 1)
    _scale(bufs[1], jE + 1)
    _scatter(1, jE + 1)
    _scatter_wait(2, jE - 2)
    _scatter_wait(3, jE - 1)
    _scatter_wait(0, jE)
    _scatter_wait(1, jE + 1)

    plsc.subcore_barrier()
    rows = pl.ds(base_row, ROWS_PER_TILE)
    pltpu.sync_copy(acc.at[rows], part_hbm.at[c].at[rows])


_sc_aggregate = functools.partial(
    pl.kernel,
    out_type=jax.ShapeDtypeStruct((NC, N, HC), jnp.float32),
    mesh=plsc.VectorSubcoreMesh(core_axis_name="c", subcore_axis_name="s"),
    compiler_params=pltpu.CompilerParams(use_tc_tiling_on_sc=False),
    scratch_types=[
        pltpu.VMEM((NB, B), jnp.int32),
        pltpu.VMEM((NB, B), jnp.int32),
        pltpu.VMEM((NB, B), jnp.float32),
        [pltpu.VMEM((B, HC), jnp.float32)] * NBUF,
        pltpu.VMEM_SHARED((N, HC), jnp.float32),
        [pltpu.SemaphoreType.DMA] * NBUF,
        [pltpu.SemaphoreType.DMA] * NBUF,
    ],
)(_sc_aggregate_body)


BM = 2000


def _tc_combine_body(p_ref, w_ref, o_ref):
    acc = (jnp.dot(p_ref[0], w_ref[:HC],
                   preferred_element_type=jnp.float32) +
           jnp.dot(p_ref[1], w_ref[HC:],
                   preferred_element_type=jnp.float32))
    o_ref[...] = jnp.maximum(acc, 0.0)


def _tc_combine(partials, W):
    return pl.pallas_call(
        _tc_combine_body,
        grid=(N // BM,),
        in_specs=[
            pl.BlockSpec((NC, BM, HC), lambda i: (0, i, 0)),
            pl.BlockSpec((M, H), lambda i: (0, 0)),
        ],
        out_specs=pl.BlockSpec((BM, H), lambda i: (i, 0)),
        out_shape=jax.ShapeDtypeStruct((N, H), jnp.float32),
    )(partials, W)


@jax.jit
def kernel(edge_index, edge_weight, X, W):
    dst = edge_index[0].reshape(NS, NB, B)
    src = edge_index[1].reshape(NS, NB, B)
    w = edge_weight.reshape(NS, NB, B)
    xt = X.reshape(N, NC, HC).swapaxes(0, 1)
    partials = _sc_aggregate(src, dst, w, xt)
    return _tc_combine(partials, W)


# speedup vs baseline: 8.8294x; 1.0671x over previous
import functools

import jax
import jax.numpy as jnp
from jax import lax
from jax.experimental import pallas as pl
from jax.experimental.pallas import tpu as pltpu
from jax.experimental.pallas import tpu_sc as plsc

N = 10000
E = 320000
M = 128
H = 128

NC = 2
NS = 16
HC = M // NC
EPT = E // NS
B = 80
NB = EPT // B
LANES = 16
HCHUNKS = HC // LANES
G16 = B // LANES
ROWS_PER_TILE = 632


NBUF = 4
NITER = 62


def _sc_aggregate_body(src_hbm, dst_hbm, w_hbm, xt_hbm, part_hbm,
                       src_v, dst_v, w_v, bufs, acc, gsems, ssems):
    c = lax.axis_index("c")
    s = lax.axis_index("s")

    pltpu.sync_copy(src_hbm.at[s], src_v)
    pltpu.sync_copy(dst_hbm.at[s], dst_v)
    pltpu.sync_copy(w_hbm.at[s], w_v)

    def _fix(j, _):
        for g in range(G16):
            sl = pl.ds(g * LANES, LANES)
            src_v[j, sl] = src_v[j, sl] * 2 + c
        return 0

    lax.fori_loop(0, NB, _fix, 0)

    zeros16 = jnp.zeros((LANES,), jnp.float32)
    g0 = bufs[0]

    def _zrow(i, _):
        for h in range(HCHUNKS):
            g0[i, pl.ds(h * LANES, LANES)] = zeros16
        return 0

    lax.fori_loop(0, B, _zrow, 0)
    base_row = pl.multiple_of(
        jnp.minimum(s * ROWS_PER_TILE, N - ROWS_PER_TILE), 8)
    full, rem = divmod(ROWS_PER_TILE, B)
    for k in range(full):
        pltpu.sync_copy(g0, acc.at[pl.ds(base_row + k * B, B)])
    if rem:
        pltpu.sync_copy(g0.at[pl.ds(0, rem)],
                        acc.at[pl.ds(base_row + full * B, rem)])
    plsc.subcore_barrier()

    xc_hbm = xt_hbm

    def _scale(buf, j):
        for g in range(G16):
            wvec = w_v[j, pl.ds(g * LANES, LANES)]
            for r in range(LANES):
                wr = wvec[r]
                row = g * LANES + r
                for h in range(HCHUNKS):
                    sl = pl.ds(h * LANES, LANES)
                    buf[row, sl] = buf[row, sl] * wr

    def _gather(p, j):
        pltpu.async_copy(xc_hbm.at[src_v.at[j]], bufs[p], gsems[p])

    def _gather_wait(p, j):
        pltpu.make_async_copy(xc_hbm.at[src_v.at[j]], bufs[p], gsems[p]).wait()

    def _scatter(p, j):
        pltpu.async_copy(bufs[p], acc.at[dst_v.at[j]], ssems[p], add=True)

    def _scatter_wait(p, j):
        pltpu.make_async_copy(bufs[p], acc.at[dst_v.at[j]], ssems[p]).wait()

    _gather(0, 0)
    _gather(1, 1)

    def _step(jj, _):
        j0 = jj * NBUF
        for p in range(NBUF):
            j = j0 + p
            q = (p + 2) % NBUF
            _gather_wait(p, j)
            _scale(bufs[p], j)
            _scatter(p, j)
            if p < 2:
                @pl.when(jj > 0)
                def _():
                    _scatter_wait(q, j - 2)
                    _gather(q, j + 2)

                @pl.when(jj == 0)
                def _():
                    _gather(q, j + 2)
            else:
                _scatter_wait(q, j - 2)
                _gather(q, j + 2)
        return 0

    lax.fori_loop(0, NITER, _step, 0)

    jE = NBUF * NITER
    _gather_wait(0, jE)
    _scale(bufs[0], jE)
    _scatter(0, jE)
    _gather_wait(1, jE + 1)
    _scale(bufs[1], jE + 1)
    _scatter(1, jE + 1)
    _scatter_wait(2, jE - 2)
    _scatter_wait(3, jE - 1)
    _scatter_wait(0, jE)
    _scatter_wait(1, jE + 1)

    plsc.subcore_barrier()
    rows = pl.ds(base_row, ROWS_PER_TILE)
    pltpu.sync_copy(acc.at[rows], part_hbm.at[c].at[rows])


_sc_aggregate = functools.partial(
    pl.kernel,
    out_type=jax.ShapeDtypeStruct((NC, N, HC), jnp.float32),
    mesh=plsc.VectorSubcoreMesh(core_axis_name="c", subcore_axis_name="s"),
    compiler_params=pltpu.CompilerParams(use_tc_tiling_on_sc=False),
    scratch_types=[
        pltpu.VMEM((NB, B), jnp.int32),
        pltpu.VMEM((NB, B), jnp.int32),
        pltpu.VMEM((NB, B), jnp.float32),
        [pltpu.VMEM((B, HC), jnp.float32)] * NBUF,
        pltpu.VMEM_SHARED((N, HC), jnp.float32),
        [pltpu.SemaphoreType.DMA] * NBUF,
        [pltpu.SemaphoreType.DMA] * NBUF,
    ],
)(_sc_aggregate_body)


BM = 2000


def _tc_combine_body(p_ref, w_ref, o_ref):
    acc = (jnp.dot(p_ref[0], w_ref[:HC],
                   preferred_element_type=jnp.float32) +
           jnp.dot(p_ref[1], w_ref[HC:],
                   preferred_element_type=jnp.float32))
    o_ref[...] = jnp.maximum(acc, 0.0)


def _tc_combine(partials, W):
    return pl.pallas_call(
        _tc_combine_body,
        grid=(N // BM,),
        in_specs=[
            pl.BlockSpec((NC, BM, HC), lambda i: (0, i, 0)),
            pl.BlockSpec((M, H), lambda i: (0, 0)),
        ],
        out_specs=pl.BlockSpec((BM, H), lambda i: (i, 0)),
        out_shape=jax.ShapeDtypeStruct((N, H), jnp.float32),
    )(partials, W)


@jax.jit
def kernel(edge_index, edge_weight, X, W):
    dst = edge_index[0].reshape(NS, NB, B)
    src = edge_index[1].reshape(NS, NB, B)
    w = edge_weight.reshape(NS, NB, B)
    x2 = X.reshape(NC * N, HC)
    partials = _sc_aggregate(src, dst, w, x2)
    return _tc_combine(partials, W)
